# Initial kernel scaffold; baseline (speedup 1.0000x reference)
import jax
import jax.numpy as jnp
from jax.experimental import pallas as pl


def kernel(x, edge_index, W1, att_src1, att_dst1, b1, W2, att_src2, att_dst2, b2):
    raise NotImplementedError("write your pallas kernel here")


# speedup vs baseline: 48.3619x; 48.3619x over previous
import functools

import jax
import jax.numpy as jnp
from jax import lax
from jax.experimental import pallas as pl
from jax.experimental.pallas import tpu as pltpu
from jax.experimental.pallas import tpu_sc as plsc

N = 10000
E = 320000
F_IN = 1433
HID = 8
HEADS = 8
NCLS = 7

NC = 2
NS = 16
NW = NC * NS
BLK = 512
Z = 10240
PAD_NODE = N
B = 128
ET = E + N
EPW = ((ET + NW * B - 1) // (NW * B)) * B
ET_PAD = EPW * NW

W1_COLS = HEADS * HID
ROW1 = 80
ROW2 = 16


def _tc1_body(x_ref, w_ref, asrc_ref, adst_ref, hs_ref, ad_ref):
    h = jnp.dot(x_ref[...], w_ref[...], preferred_element_type=jnp.float32)
    h3 = h.reshape(BLK, HEADS, HID)
    a_s = (h3 * asrc_ref[...][None, :, :]).sum(-1)
    a_d = (h3 * adst_ref[...][None, :, :]).sum(-1)
    zpad = jnp.zeros((BLK, 8), jnp.float32)
    hs_ref[...] = jnp.concatenate([h, a_s, zpad], axis=1)
    ad_ref[...] = jnp.concatenate([a_d, zpad], axis=1)


def _tc1(x_pad, W1, att_src1, att_dst1):
    grid = (Z // BLK,)
    return pl.pallas_call(
        _tc1_body,
        grid=grid,
        in_specs=[
            pl.BlockSpec((BLK, F_IN), lambda i: (i, 0)),
            pl.BlockSpec((F_IN, W1_COLS), lambda i: (0, 0)),
            pl.BlockSpec((HEADS, HID), lambda i: (0, 0)),
            pl.BlockSpec((HEADS, HID), lambda i: (0, 0)),
        ],
        out_specs=[
            pl.BlockSpec((BLK, ROW1), lambda i: (i, 0)),
            pl.BlockSpec((BLK, 16), lambda i: (i, 0)),
        ],
        out_shape=[
            jax.ShapeDtypeStruct((Z, ROW1), jnp.float32),
            jax.ShapeDtypeStruct((Z, 16), jnp.float32),
        ],
    )(x_pad, W1, att_src1, att_dst1)


def _lane_iota():
    return lax.iota(jnp.int32, 16)


def _bcast16(v, idx):
    return lax.gather(
        v,
        idx[:, None],
        lax.GatherDimensionNumbers(
            offset_dims=(), collapsed_slice_dims=(0,), start_index_map=(0,)),
        (1,),
        mode=lax.GatherScatterMode.PROMISE_IN_BOUNDS,
    )


def _leaky_exp(s):
    return jnp.exp(jnp.where(s >= 0.0, s, 0.2 * s))


def _sc1_body(hs_hbm, ad_hbm, src_hbm, dst_hbm, zero_hbm, out_hbm,
              src_idx, dst_idx, hs_rows, ad_rows, out_rows, acc,
              sem_a, sem_b):
    cid = lax.axis_index("c")
    sid = lax.axis_index("s")
    wid = sid * NC + cid
    zrows = Z // NS
    pltpu.sync_copy(zero_hbm.at[pl.ds(sid * zrows, zrows)],
                    acc.at[pl.ds(sid * zrows, zrows)])
    plsc.subcore_barrier()

    lane = _lane_iota()
    half = (lane >= 8).astype(jnp.int32)

    base = wid * EPW

    def chunk(ci, carry):
        off = base + ci * B
        pltpu.sync_copy(src_hbm.at[pl.ds(off, B)], src_idx)
        pltpu.sync_copy(dst_hbm.at[pl.ds(off, B)], dst_idx)
        cp_a = pltpu.async_copy(hs_hbm.at[src_idx], hs_rows, sem_a)
        cp_b = pltpu.async_copy(ad_hbm.at[dst_idx], ad_rows, sem_b)
        cp_a.wait()
        cp_b.wait()

        def edge(b, c2):
            va = hs_rows[b, pl.ds(64, 16)]
            vd = ad_rows[b, pl.ds(0, 16)]
            ex = _leaky_exp(va + vd)
            out_rows[b, pl.ds(64, 16)] = ex
            for j in range(4):
                m = _bcast16(ex, 2 * j + half)
                out_rows[b, pl.ds(16 * j, 16)] = hs_rows[b, pl.ds(16 * j, 16)] * m
            return c2

        lax.fori_loop(0, B, edge, 0)
        pltpu.sync_copy(out_rows, acc.at[dst_idx], add=True)
        return carry

    lax.fori_loop(0, EPW // B, chunk, 0)
    plsc.subcore_barrier()
    pltpu.sync_copy(acc.at[pl.ds(sid * zrows, zrows)],
                    out_hbm.at[cid, pl.ds(sid * zrows, zrows)])


def _sc1(hs, ad, src, dst, zero80):
    mesh = plsc.VectorSubcoreMesh(core_axis_name="c", subcore_axis_name="s")
    f = pl.kernel(
        _sc1_body,
        out_type=jax.ShapeDtypeStruct((NC, Z, ROW1), jnp.float32),
        mesh=mesh,
        compiler_params=pltpu.CompilerParams(
            use_tc_tiling_on_sc=False, needs_layout_passes=False),
        scratch_types=[
            pltpu.VMEM((B,), jnp.int32),
            pltpu.VMEM((B,), jnp.int32),
            pltpu.VMEM((B, ROW1), jnp.float32),
            pltpu.VMEM((B, 16), jnp.float32),
            pltpu.VMEM((B, ROW1), jnp.float32),
            pltpu.VMEM_SHARED((Z, ROW1), jnp.float32),
            pltpu.SemaphoreType.DMA,
            pltpu.SemaphoreType.DMA,
        ],
    )
    return f(hs, ad, src, dst, zero80)


def _tc2_body(parts_ref, b1_ref, w2_ref, as2_ref, ad2_ref, hs2_ref, ad2o_ref):
    p = parts_ref[...]
    tot = p[0] + p[1]
    numer = tot[:, :W1_COLS].reshape(BLK, HEADS, HID)
    denom = tot[:, W1_COLS:W1_COLS + HEADS]
    h1 = numer / (denom[:, :, None] + 1e-16)
    h1 = h1.reshape(BLK, W1_COLS) + b1_ref[...][None, :]
    h1 = jnp.where(h1 > 0.0, h1, jnp.exp(jnp.minimum(h1, 0.0)) - 1.0)
    h2 = jnp.dot(h1, w2_ref[...], preferred_element_type=jnp.float32)
    a_s2 = (h2 * as2_ref[...]).sum(-1, keepdims=True)
    a_d2 = (h2 * ad2_ref[...]).sum(-1, keepdims=True)
    hs2_ref[...] = jnp.concatenate(
        [h2, a_s2, jnp.zeros((BLK, 8), jnp.float32)], axis=1)
    ad2o_ref[...] = jnp.broadcast_to(a_d2, (BLK, 16))


def _tc2(parts, b1, W2, att_src2, att_dst2):
    grid = (Z // BLK,)
    return pl.pallas_call(
        _tc2_body,
        grid=grid,
        in_specs=[
            pl.BlockSpec((NC, BLK, ROW1), lambda i: (0, i, 0)),
            pl.BlockSpec((W1_COLS,), lambda i: (0,)),
            pl.BlockSpec((W1_COLS, NCLS), lambda i: (0, 0)),
            pl.BlockSpec((1, NCLS), lambda i: (0, 0)),
            pl.BlockSpec((1, NCLS), lambda i: (0, 0)),
        ],
        out_specs=[
            pl.BlockSpec((BLK, ROW2), lambda i: (i, 0)),
            pl.BlockSpec((BLK, ROW2), lambda i: (i, 0)),
        ],
        out_shape=[
            jax.ShapeDtypeStruct((Z, ROW2), jnp.float32),
            jax.ShapeDtypeStruct((Z, ROW2), jnp.float32),
        ],
    )(parts, b1, W2, att_src2, att_dst2)


def _sc2_body(hs_hbm, ad_hbm, src_hbm, dst_hbm, zero_hbm, out_hbm,
              src_idx, dst_idx, hs_rows, ad_rows, out_rows, acc,
              sem_a, sem_b):
    cid = lax.axis_index("c")
    sid = lax.axis_index("s")
    wid = sid * NC + cid
    zrows = Z // NS
    pltpu.sync_copy(zero_hbm.at[pl.ds(sid * zrows, zrows)],
                    acc.at[pl.ds(sid * zrows, zrows)])
    plsc.subcore_barrier()

    lane = _lane_iota()
    seven = jnp.full((16,), 7, jnp.int32)
    is_den = lane == 7

    base = wid * EPW

    def chunk(ci, carry):
        off = base + ci * B
        pltpu.sync_copy(src_hbm.at[pl.ds(off, B)], src_idx)
        pltpu.sync_copy(dst_hbm.at[pl.ds(off, B)], dst_idx)
        cp_a = pltpu.async_copy(hs_hbm.at[src_idx], hs_rows, sem_a)
        cp_b = pltpu.async_copy(ad_hbm.at[dst_idx], ad_rows, sem_b)
        cp_a.wait()
        cp_b.wait()

        def edge(b, c2):
            vh = hs_rows[b, pl.ds(0, 16)]
            vad = ad_rows[b, pl.ds(0, 16)]
            asb = _bcast16(vh, seven)
            alpha = _leaky_exp(asb + vad)
            out_rows[b, pl.ds(0, 16)] = jnp.where(is_den, alpha, vh * alpha)
            return c2

        lax.fori_loop(0, B, edge, 0)
        pltpu.sync_copy(out_rows, acc.at[dst_idx], add=True)
        return carry

    lax.fori_loop(0, EPW // B, chunk, 0)
    plsc.subcore_barrier()
    pltpu.sync_copy(acc.at[pl.ds(sid * zrows, zrows)],
                    out_hbm.at[cid, pl.ds(sid * zrows, zrows)])


def _sc2(hs2, ad2, src, dst, zero16):
    mesh = plsc.VectorSubcoreMesh(core_axis_name="c", subcore_axis_name="s")
    f = pl.kernel(
        _sc2_body,
        out_type=jax.ShapeDtypeStruct((NC, Z, ROW2), jnp.float32),
        mesh=mesh,
        compiler_params=pltpu.CompilerParams(
            use_tc_tiling_on_sc=False, needs_layout_passes=False),
        scratch_types=[
            pltpu.VMEM((B,), jnp.int32),
            pltpu.VMEM((B,), jnp.int32),
            pltpu.VMEM((B, ROW2), jnp.float32),
            pltpu.VMEM((B, ROW2), jnp.float32),
            pltpu.VMEM((B, ROW2), jnp.float32),
            pltpu.VMEM_SHARED((Z, ROW2), jnp.float32),
            pltpu.SemaphoreType.DMA,
            pltpu.SemaphoreType.DMA,
        ],
    )
    return f(hs2, ad2, src, dst, zero16)


def _tc3_body(parts_ref, b2_ref, out_ref):
    p = parts_ref[...]
    tot = p[0] + p[1]
    numer = tot[:, :NCLS]
    denom = tot[:, NCLS:NCLS + 1]
    res = numer / (denom + 1e-16) + b2_ref[...][None, :]
    out_ref[...] = jnp.concatenate(
        [res, jnp.zeros((BLK, ROW2 - NCLS), jnp.float32)], axis=1)


def _tc3(parts2, b2):
    grid = (Z // BLK,)
    return pl.pallas_call(
        _tc3_body,
        grid=grid,
        in_specs=[
            pl.BlockSpec((NC, BLK, ROW2), lambda i: (0, i, 0)),
            pl.BlockSpec((NCLS,), lambda i: (0,)),
        ],
        out_specs=pl.BlockSpec((BLK, ROW2), lambda i: (i, 0)),
        out_shape=jax.ShapeDtypeStruct((Z, ROW2), jnp.float32),
    )(parts2, b2)


def kernel(x, edge_index, W1, att_src1, att_dst1, b1, W2, att_src2,
           att_dst2, b2):
    x_pad = jnp.pad(x, ((0, Z - N), (0, 0)))
    loop = jnp.arange(N, dtype=jnp.int32)
    padv = jnp.full((ET_PAD - ET,), PAD_NODE, jnp.int32)
    src = jnp.concatenate([edge_index[0].astype(jnp.int32), loop, padv])
    dst = jnp.concatenate([edge_index[1].astype(jnp.int32), loop, padv])
    zero80 = jnp.zeros((Z, ROW1), jnp.float32)
    zero16 = jnp.zeros((Z, ROW2), jnp.float32)

    hs1, ad1 = _tc1(x_pad, W1, att_src1, att_dst1)
    parts1 = _sc1(hs1, ad1, src, dst, zero80)
    hs2, ad2 = _tc2(parts1, b1, W2, att_src2, att_dst2)
    parts2 = _sc2(hs2, ad2, src, dst, zero16)
    out = _tc3(parts2, b2)
    return out[:N, :NCLS]


# speedup vs baseline: 68.3401x; 1.4131x over previous
import functools

import jax
import jax.numpy as jnp
from jax import lax
from jax.experimental import pallas as pl
from jax.experimental.pallas import tpu as pltpu
from jax.experimental.pallas import tpu_sc as plsc

N = 10000
E = 320000
F_IN = 1433
HID = 8
HEADS = 8
NCLS = 7

NC = 2
NS = 16
NW = NC * NS
BLK = 512
Z = 10240
PAD_NODE = N
B = 128
ET = E + N
CHUNKS = -(-ET // (NW * B))
CHUNKS += CHUNKS % 2
EPW = CHUNKS * B
ET_PAD = EPW * NW

W1_COLS = HEADS * HID
ROW1 = 80
ROW2 = 16

_SC_PARAMS = pltpu.CompilerParams(
    use_tc_tiling_on_sc=False, needs_layout_passes=False)


def _tc1_body(x_ref, w_ref, asrc_ref, adst_ref, hs_ref, ad_ref):
    h = jnp.dot(x_ref[...], w_ref[...], preferred_element_type=jnp.float32)
    h3 = h.reshape(BLK, HEADS, HID)
    a_s = (h3 * asrc_ref[...][None, :, :]).sum(-1)
    a_d = (h3 * adst_ref[...][None, :, :]).sum(-1)
    zpad = jnp.zeros((BLK, 8), jnp.float32)
    hs_ref[...] = jnp.concatenate([h, a_s, zpad], axis=1)
    ad_ref[...] = jnp.concatenate([a_d, zpad], axis=1)


def _tc1(x_pad, W1, att_src1, att_dst1):
    return pl.pallas_call(
        _tc1_body,
        grid=(Z // BLK,),
        in_specs=[
            pl.BlockSpec((BLK, F_IN), lambda i: (i, 0)),
            pl.BlockSpec((F_IN, W1_COLS), lambda i: (0, 0)),
            pl.BlockSpec((HEADS, HID), lambda i: (0, 0)),
            pl.BlockSpec((HEADS, HID), lambda i: (0, 0)),
        ],
        out_specs=[
            pl.BlockSpec((BLK, ROW1), lambda i: (i, 0)),
            pl.BlockSpec((BLK, 16), lambda i: (i, 0)),
        ],
        out_shape=[
            jax.ShapeDtypeStruct((Z, ROW1), jnp.float32),
            jax.ShapeDtypeStruct((Z, 16), jnp.float32),
        ],
    )(x_pad, W1, att_src1, att_dst1)


def _bcast16(v, idx):
    return lax.gather(
        v,
        idx[:, None],
        lax.GatherDimensionNumbers(
            offset_dims=(), collapsed_slice_dims=(0,), start_index_map=(0,)),
        (1,),
        mode=lax.GatherScatterMode.PROMISE_IN_BOUNDS,
    )


def _leaky_exp(s):
    return jnp.exp(jnp.where(s >= 0.0, s, 0.2 * s))


def _sc_edge_kernel(row_w, edge_fn):
    def body(hs_hbm, ad_hbm, src_hbm, dst_hbm, zero_hbm, out_hbm,
             src_all, dst_all, hs_rows, ad_rows, out_rows,
             acc, sem_a0, sem_a1, sem_b0, sem_b1):
        cid = lax.axis_index("c")
        sid = lax.axis_index("s")
        wid = sid * NC + cid
        zrows = Z // NS
        pltpu.sync_copy(zero_hbm.at[pl.ds(sid * zrows, zrows)],
                        acc.at[pl.ds(sid * zrows, zrows)])
        pltpu.sync_copy(src_hbm.at[pl.ds(wid * CHUNKS, CHUNKS)],
                        src_all.at[pl.ds(0, CHUNKS)])
        pltpu.sync_copy(dst_hbm.at[pl.ds(wid * CHUNKS, CHUNKS)],
                        dst_all.at[pl.ds(0, CHUNKS)])
        pltpu.sync_copy(src_hbm.at[pl.ds(wid * CHUNKS, 2)],
                        src_all.at[pl.ds(CHUNKS, 2)])
        pltpu.sync_copy(dst_hbm.at[pl.ds(wid * CHUNKS, 2)],
                        dst_all.at[pl.ds(CHUNKS, 2)])
        plsc.subcore_barrier()

        sems_a = (sem_a0, sem_a1)
        sems_b = (sem_b0, sem_b1)

        def gather_start(ci, p):
            ca = pltpu.async_copy(hs_hbm.at[src_all.at[ci]],
                                  hs_rows.at[p], sems_a[p])
            cb = pltpu.async_copy(ad_hbm.at[dst_all.at[ci]],
                                  ad_rows.at[p], sems_b[p])
            return ca, cb

        def gather_wait(p):
            pltpu.make_async_copy(hs_hbm.at[src_all.at[0]],
                                  hs_rows.at[p], sems_a[p]).wait()
            pltpu.make_async_copy(ad_hbm.at[dst_all.at[0]],
                                  ad_rows.at[p], sems_b[p]).wait()

        gather_start(0, 0)
        gather_start(1, 1)

        def outer(i, carry):
            ci0 = 2 * i
            for p in range(2):
                ci = ci0 + p
                gather_wait(p)
                plsc.parallel_loop(0, B, 1, unroll=4)(
                    functools.partial(edge_fn, hs_rows.at[p], ad_rows.at[p],
                                      out_rows))
                gather_start(ci + 2, p)
                pltpu.sync_copy(out_rows, acc.at[dst_all.at[ci]], add=True)
            return carry

        lax.fori_loop(0, CHUNKS // 2, outer, 0)
        gather_wait(0)
        gather_wait(1)
        plsc.subcore_barrier()
        pltpu.sync_copy(acc.at[pl.ds(sid * zrows, zrows)],
                        out_hbm.at[cid, pl.ds(sid * zrows, zrows)])

    def make(hs, ad, src2, dst2, zero):
        mesh = plsc.VectorSubcoreMesh(core_axis_name="c", subcore_axis_name="s")
        f = pl.kernel(
            body,
            out_type=jax.ShapeDtypeStruct((NC, Z, row_w), jnp.float32),
            mesh=mesh,
            compiler_params=_SC_PARAMS,
            scratch_types=[
                pltpu.VMEM((CHUNKS + 2, B), jnp.int32),
                pltpu.VMEM((CHUNKS + 2, B), jnp.int32),
                pltpu.VMEM((2, B, row_w), jnp.float32),
                pltpu.VMEM((2, B, 16), jnp.float32),
                pltpu.VMEM((B, row_w), jnp.float32),
                pltpu.VMEM_SHARED((Z, row_w), jnp.float32),
                pltpu.SemaphoreType.DMA,
                pltpu.SemaphoreType.DMA,
                pltpu.SemaphoreType.DMA,
                pltpu.SemaphoreType.DMA,
            ],
        )
        return f(hs, ad, src2, dst2, zero)

    return make


def _edge1(hs_rows, ad_rows, out_rows, b):
    lane = lax.iota(jnp.int32, 16)
    half = (lane >= 8).astype(jnp.int32)
    va = hs_rows[b, pl.ds(64, 16)]
    vd = ad_rows[b, pl.ds(0, 16)]
    ex = _leaky_exp(va + vd)
    out_rows[b, pl.ds(64, 16)] = ex
    for j in range(4):
        m = _bcast16(ex, 2 * j + half)
        out_rows[b, pl.ds(16 * j, 16)] = hs_rows[b, pl.ds(16 * j, 16)] * m


def _edge2(hs_rows, ad_rows, out_rows, b):
    lane = lax.iota(jnp.int32, 16)
    seven = jnp.full((16,), 7, jnp.int32)
    vh = hs_rows[b, pl.ds(0, 16)]
    vad = ad_rows[b, pl.ds(0, 16)]
    asb = _bcast16(vh, seven)
    alpha = _leaky_exp(asb + vad)
    out_rows[b, pl.ds(0, 16)] = jnp.where(lane == 7, alpha, vh * alpha)


_sc1 = _sc_edge_kernel(ROW1, _edge1)
_sc2 = _sc_edge_kernel(ROW2, _edge2)


def _tc2_body(parts_ref, b1_ref, w2_ref, as2_ref, ad2_ref, hs2_ref, ad2o_ref):
    p = parts_ref[...]
    tot = p[0] + p[1]
    numer = tot[:, :W1_COLS].reshape(BLK, HEADS, HID)
    denom = tot[:, W1_COLS:W1_COLS + HEADS]
    h1 = numer / (denom[:, :, None] + 1e-16)
    h1 = h1.reshape(BLK, W1_COLS) + b1_ref[...][None, :]
    h1 = jnp.where(h1 > 0.0, h1, jnp.exp(jnp.minimum(h1, 0.0)) - 1.0)
    h2 = jnp.dot(h1, w2_ref[...], preferred_element_type=jnp.float32)
    a_s2 = (h2 * as2_ref[...]).sum(-1, keepdims=True)
    a_d2 = (h2 * ad2_ref[...]).sum(-1, keepdims=True)
    hs2_ref[...] = jnp.concatenate(
        [h2, a_s2, jnp.zeros((BLK, 8), jnp.float32)], axis=1)
    ad2o_ref[...] = jnp.broadcast_to(a_d2, (BLK, 16))


def _tc2(parts, b1, W2, att_src2, att_dst2):
    return pl.pallas_call(
        _tc2_body,
        grid=(Z // BLK,),
        in_specs=[
            pl.BlockSpec((NC, BLK, ROW1), lambda i: (0, i, 0)),
            pl.BlockSpec((W1_COLS,), lambda i: (0,)),
            pl.BlockSpec((W1_COLS, NCLS), lambda i: (0, 0)),
            pl.BlockSpec((1, NCLS), lambda i: (0, 0)),
            pl.BlockSpec((1, NCLS), lambda i: (0, 0)),
        ],
        out_specs=[
            pl.BlockSpec((BLK, ROW2), lambda i: (i, 0)),
            pl.BlockSpec((BLK, ROW2), lambda i: (i, 0)),
        ],
        out_shape=[
            jax.ShapeDtypeStruct((Z, ROW2), jnp.float32),
            jax.ShapeDtypeStruct((Z, ROW2), jnp.float32),
        ],
    )(parts, b1, W2, att_src2, att_dst2)


def _tc3_body(parts_ref, b2_ref, out_ref):
    p = parts_ref[...]
    tot = p[0] + p[1]
    numer = tot[:, :NCLS]
    denom = tot[:, NCLS:NCLS + 1]
    res = numer / (denom + 1e-16) + b2_ref[...][None, :]
    out_ref[...] = jnp.concatenate(
        [res, jnp.zeros((BLK, ROW2 - NCLS), jnp.float32)], axis=1)


def _tc3(parts2, b2):
    return pl.pallas_call(
        _tc3_body,
        grid=(Z // BLK,),
        in_specs=[
            pl.BlockSpec((NC, BLK, ROW2), lambda i: (0, i, 0)),
            pl.BlockSpec((NCLS,), lambda i: (0,)),
        ],
        out_specs=pl.BlockSpec((BLK, ROW2), lambda i: (i, 0)),
        out_shape=jax.ShapeDtypeStruct((Z, ROW2), jnp.float32),
    )(parts2, b2)


def kernel(x, edge_index, W1, att_src1, att_dst1, b1, W2, att_src2,
           att_dst2, b2):
    x_pad = jnp.pad(x, ((0, Z - N), (0, 0)))
    loop = jnp.arange(N, dtype=jnp.int32)
    padv = jnp.full((ET_PAD - ET,), PAD_NODE, jnp.int32)
    src = jnp.concatenate([edge_index[0].astype(jnp.int32), loop, padv])
    dst = jnp.concatenate([edge_index[1].astype(jnp.int32), loop, padv])
    src2 = src.reshape(NW * CHUNKS, B)
    dst2 = dst.reshape(NW * CHUNKS, B)
    zero80 = jnp.zeros((Z, ROW1), jnp.float32)
    zero16 = jnp.zeros((Z, ROW2), jnp.float32)

    hs1, ad1 = _tc1(x_pad, W1, att_src1, att_dst1)
    parts1 = _sc1(hs1, ad1, src2, dst2, zero80)
    hs2, ad2 = _tc2(parts1, b1, W2, att_src2, att_dst2)
    parts2 = _sc2(hs2, ad2, src2, dst2, zero16)
    out = _tc3(parts2, b2)
    return out[:N, :NCLS]


# speedup vs baseline: 103.7293x; 1.5178x over previous
import functools

import jax
import jax.numpy as jnp
from jax import lax
from jax.experimental import pallas as pl
from jax.experimental.pallas import tpu as pltpu
from jax.experimental.pallas import tpu_sc as plsc

N = 10000
E = 320000
F_IN = 1433
HID = 8
HEADS = 8
NCLS = 7

NC = 2
NS = 16
NW = NC * NS
BLK = 512
Z = 10240
PAD_NODE = N
B = 128
ET = E + N
CHUNKS = -(-ET // (NW * B))
CHUNKS += CHUNKS % 2
EPW = CHUNKS * B
ET_PAD = EPW * NW

W1_COLS = HEADS * HID
ROW1 = 80
ROW2 = 16

_SC_PARAMS = pltpu.CompilerParams(
    use_tc_tiling_on_sc=False, needs_layout_passes=False)


def _tc1_body(xt_ref, w_ref, asrc_ref, adst_ref, hs_ref, ad_ref):
    h = lax.dot_general(xt_ref[...], w_ref[...], (((0,), (0,)), ((), ())),
                        preferred_element_type=jnp.float32)
    h3 = h.reshape(BLK, HEADS, HID)
    a_s = (h3 * asrc_ref[...][None, :, :]).sum(-1)
    a_d = (h3 * adst_ref[...][None, :, :]).sum(-1)
    zpad = jnp.zeros((BLK, 8), jnp.float32)
    rows = pl.program_id(0) * BLK + lax.broadcasted_iota(jnp.int32, (BLK, 1), 0)
    valid = rows < N
    hs_ref[...] = jnp.where(valid, jnp.concatenate([h, a_s, zpad], axis=1), 0.0)
    ad_ref[...] = jnp.where(valid, jnp.concatenate([a_d, zpad], axis=1), 0.0)


def _tc1(xt, W1, att_src1, att_dst1):
    return pl.pallas_call(
        _tc1_body,
        grid=(Z // BLK,),
        in_specs=[
            pl.BlockSpec((F_IN, BLK), lambda i: (0, i)),
            pl.BlockSpec((F_IN, W1_COLS), lambda i: (0, 0)),
            pl.BlockSpec((HEADS, HID), lambda i: (0, 0)),
            pl.BlockSpec((HEADS, HID), lambda i: (0, 0)),
        ],
        out_specs=[
            pl.BlockSpec((BLK, ROW1), lambda i: (i, 0)),
            pl.BlockSpec((BLK, 16), lambda i: (i, 0)),
        ],
        out_shape=[
            jax.ShapeDtypeStruct((Z, ROW1), jnp.float32),
            jax.ShapeDtypeStruct((Z, 16), jnp.float32),
        ],
    )(xt, W1, att_src1, att_dst1)


def _bcast16(v, idx):
    return lax.gather(
        v,
        idx[:, None],
        lax.GatherDimensionNumbers(
            offset_dims=(), collapsed_slice_dims=(0,), start_index_map=(0,)),
        (1,),
        mode=lax.GatherScatterMode.PROMISE_IN_BOUNDS,
    )


def _leaky_exp(s):
    return jnp.exp(jnp.where(s >= 0.0, s, 0.2 * s))


def _sc_edge_kernel(row_w, edge_fn):
    def body(hs_hbm, ad_hbm, src_hbm, dst_hbm, zero_hbm, out_hbm,
             src_all, dst_all, hs_rows, ad_rows, out_rows,
             acc, sem_a0, sem_a1, sem_b0, sem_b1, sem_s0, sem_s1):
        cid = lax.axis_index("c")
        sid = lax.axis_index("s")
        wid = sid * NC + cid
        zrows = Z // NS
        pltpu.sync_copy(zero_hbm.at[pl.ds(sid * zrows, zrows)],
                        acc.at[pl.ds(sid * zrows, zrows)])
        pltpu.sync_copy(src_hbm.at[pl.ds(wid * CHUNKS, CHUNKS)],
                        src_all.at[pl.ds(0, CHUNKS)])
        pltpu.sync_copy(dst_hbm.at[pl.ds(wid * CHUNKS, CHUNKS)],
                        dst_all.at[pl.ds(0, CHUNKS)])
        pltpu.sync_copy(src_hbm.at[pl.ds(wid * CHUNKS, 2)],
                        src_all.at[pl.ds(CHUNKS, 2)])
        pltpu.sync_copy(dst_hbm.at[pl.ds(wid * CHUNKS, 2)],
                        dst_all.at[pl.ds(CHUNKS, 2)])
        pltpu.sync_copy(zero_hbm.at[pl.ds(0, B)], out_rows.at[0])
        pltpu.sync_copy(zero_hbm.at[pl.ds(0, B)], out_rows.at[1])
        plsc.subcore_barrier()

        sems_a = (sem_a0, sem_a1)
        sems_b = (sem_b0, sem_b1)
        sems_s = (sem_s0, sem_s1)

        def gather_start(ci, p):
            pltpu.async_copy(hs_hbm.at[src_all.at[ci]],
                             hs_rows.at[p], sems_a[p])
            pltpu.async_copy(ad_hbm.at[dst_all.at[ci]],
                             ad_rows.at[p], sems_b[p])

        def gather_wait(p):
            pltpu.make_async_copy(hs_hbm.at[src_all.at[0]],
                                  hs_rows.at[p], sems_a[p]).wait()
            pltpu.make_async_copy(ad_hbm.at[dst_all.at[0]],
                                  ad_rows.at[p], sems_b[p]).wait()

        def scatter_start(ci, p):
            pltpu.async_copy(out_rows.at[p], acc.at[dst_all.at[ci]],
                             sems_s[p], add=True)

        def scatter_wait(p):
            pltpu.make_async_copy(out_rows.at[p], acc.at[dst_all.at[0]],
                                  sems_s[p]).wait()

        gather_start(0, 0)
        gather_start(1, 1)
        scatter_start(0, 0)
        scatter_start(0, 1)

        def outer(i, carry):
            ci0 = 2 * i
            for p in range(2):
                ci = ci0 + p
                gather_wait(p)
                scatter_wait(p)
                plsc.parallel_loop(0, B, 1, unroll=4)(
                    functools.partial(edge_fn, hs_rows.at[p], ad_rows.at[p],
                                      out_rows.at[p]))
                gather_start(ci + 2, p)
                scatter_start(ci, p)
            return carry

        lax.fori_loop(0, CHUNKS // 2, outer, 0)
        gather_wait(0)
        gather_wait(1)
        scatter_wait(0)
        scatter_wait(1)
        plsc.subcore_barrier()
        pltpu.sync_copy(acc.at[pl.ds(sid * zrows, zrows)],
                        out_hbm.at[cid, pl.ds(sid * zrows, zrows)])

    def make(hs, ad, src2, dst2, zero):
        mesh = plsc.VectorSubcoreMesh(core_axis_name="c", subcore_axis_name="s")
        f = pl.kernel(
            body,
            out_type=jax.ShapeDtypeStruct((NC, Z, row_w), jnp.float32),
            mesh=mesh,
            compiler_params=_SC_PARAMS,
            scratch_types=[
                pltpu.VMEM((CHUNKS + 2, B), jnp.int32),
                pltpu.VMEM((CHUNKS + 2, B), jnp.int32),
                pltpu.VMEM((2, B, row_w), jnp.float32),
                pltpu.VMEM((2, B, 16), jnp.float32),
                pltpu.VMEM((2, B, row_w), jnp.float32),
                pltpu.VMEM_SHARED((Z, row_w), jnp.float32),
                pltpu.SemaphoreType.DMA,
                pltpu.SemaphoreType.DMA,
                pltpu.SemaphoreType.DMA,
                pltpu.SemaphoreType.DMA,
                pltpu.SemaphoreType.DMA,
                pltpu.SemaphoreType.DMA,
            ],
        )
        return f(hs, ad, src2, dst2, zero)

    return make


def _edge1(hs_rows, ad_rows, out_rows, b):
    lane = lax.iota(jnp.int32, 16)
    half = (lane >= 8).astype(jnp.int32)
    va = hs_rows[b, pl.ds(64, 16)]
    vd = ad_rows[b, pl.ds(0, 16)]
    ex = _leaky_exp(va + vd)
    out_rows[b, pl.ds(64, 16)] = ex
    for j in range(4):
        m = _bcast16(ex, 2 * j + half)
        out_rows[b, pl.ds(16 * j, 16)] = hs_rows[b, pl.ds(16 * j, 16)] * m


def _edge2(hs_rows, ad_rows, out_rows, b):
    lane = lax.iota(jnp.int32, 16)
    seven = jnp.full((16,), 7, jnp.int32)
    vh = hs_rows[b, pl.ds(0, 16)]
    vad = ad_rows[b, pl.ds(0, 16)]
    asb = _bcast16(vh, seven)
    alpha = _leaky_exp(asb + vad)
    out_rows[b, pl.ds(0, 16)] = jnp.where(lane == 7, alpha, vh * alpha)


_sc1 = _sc_edge_kernel(ROW1, _edge1)
_sc2 = _sc_edge_kernel(ROW2, _edge2)


def _tc2_body(parts_ref, b1_ref, w2_ref, as2_ref, ad2_ref, hs2_ref, ad2o_ref):
    p = parts_ref[...]
    tot = p[0] + p[1]
    numer = tot[:, :W1_COLS].reshape(BLK, HEADS, HID)
    denom = tot[:, W1_COLS:W1_COLS + HEADS]
    h1 = numer / (denom[:, :, None] + 1e-16)
    h1 = h1.reshape(BLK, W1_COLS) + b1_ref[...][None, :]
    h1 = jnp.where(h1 > 0.0, h1, jnp.exp(jnp.minimum(h1, 0.0)) - 1.0)
    h2 = jnp.dot(h1, w2_ref[...], preferred_element_type=jnp.float32)
    a_s2 = (h2 * as2_ref[...]).sum(-1, keepdims=True)
    a_d2 = (h2 * ad2_ref[...]).sum(-1, keepdims=True)
    hs2_ref[...] = jnp.concatenate(
        [h2, a_s2, jnp.zeros((BLK, 8), jnp.float32)], axis=1)
    ad2o_ref[...] = jnp.broadcast_to(a_d2, (BLK, 16))


def _tc2(parts, b1, W2, att_src2, att_dst2):
    return pl.pallas_call(
        _tc2_body,
        grid=(Z // BLK,),
        in_specs=[
            pl.BlockSpec((NC, BLK, ROW1), lambda i: (0, i, 0)),
            pl.BlockSpec((W1_COLS,), lambda i: (0,)),
            pl.BlockSpec((W1_COLS, NCLS), lambda i: (0, 0)),
            pl.BlockSpec((1, NCLS), lambda i: (0, 0)),
            pl.BlockSpec((1, NCLS), lambda i: (0, 0)),
        ],
        out_specs=[
            pl.BlockSpec((BLK, ROW2), lambda i: (i, 0)),
            pl.BlockSpec((BLK, ROW2), lambda i: (i, 0)),
        ],
        out_shape=[
            jax.ShapeDtypeStruct((Z, ROW2), jnp.float32),
            jax.ShapeDtypeStruct((Z, ROW2), jnp.float32),
        ],
    )(parts, b1, W2, att_src2, att_dst2)


def _tc3_body(parts_ref, b2_ref, out_ref):
    p = parts_ref[...]
    tot = p[0] + p[1]
    numer = tot[:, :NCLS]
    denom = tot[:, NCLS:NCLS + 1]
    res = numer / (denom + 1e-16) + b2_ref[...][None, :]
    out_ref[...] = jnp.concatenate(
        [res, jnp.zeros((BLK, ROW2 - NCLS), jnp.float32)], axis=1)


def _tc3(parts2, b2):
    return pl.pallas_call(
        _tc3_body,
        grid=(Z // BLK,),
        in_specs=[
            pl.BlockSpec((NC, BLK, ROW2), lambda i: (0, i, 0)),
            pl.BlockSpec((NCLS,), lambda i: (0,)),
        ],
        out_specs=pl.BlockSpec((BLK, ROW2), lambda i: (i, 0)),
        out_shape=jax.ShapeDtypeStruct((Z, ROW2), jnp.float32),
    )(parts2, b2)


def kernel(x, edge_index, W1, att_src1, att_dst1, b1, W2, att_src2,
           att_dst2, b2):
    loop = jnp.arange(N, dtype=jnp.int32)
    padv = jnp.full((ET_PAD - ET,), PAD_NODE, jnp.int32)
    src = jnp.concatenate([edge_index[0].astype(jnp.int32), loop, padv])
    dst = jnp.concatenate([edge_index[1].astype(jnp.int32), loop, padv])
    src2 = src.reshape(NW * CHUNKS, B)
    dst2 = dst.reshape(NW * CHUNKS, B)
    zero80 = jnp.zeros((Z, ROW1), jnp.float32)
    zero16 = jnp.zeros((Z, ROW2), jnp.float32)

    hs1, ad1 = _tc1(jnp.transpose(x), W1, att_src1, att_dst1)
    parts1 = _sc1(hs1, ad1, src2, dst2, zero80)
    hs2, ad2 = _tc2(parts1, b1, W2, att_src2, att_dst2)
    parts2 = _sc2(hs2, ad2, src2, dst2, zero16)
    out = _tc3(parts2, b2)
    return out[:N, :NCLS]
